# baseline (speedup 1.0000x reference)
import functools

import jax
import jax.numpy as jnp
from jax import lax
from jax.experimental import pallas as pl
from jax.experimental.pallas import tpu as pltpu
from jax.experimental.pallas import tpu_sc as plsc

B, L, D, V = 1024, 200, 64, 1000000
N = B * L
NC, NS, LANES = 2, 16, 16
NW = NC * NS
PER_W = N // NW
CH = 128
NCH = PER_W // CH

@functools.lru_cache(maxsize=1)
def _make_gather_combine():
    mesh = plsc.VectorSubcoreMesh(
        core_axis_name="c", subcore_axis_name="s",
        num_cores=NC, num_subcores=NS)

    @functools.partial(
        pl.kernel,
        mesh=mesh,
        compiler_params=pltpu.CompilerParams(use_tc_tiling_on_sc=False),
        out_type=jax.ShapeDtypeStruct((N, D), jnp.float32),
        scratch_types=[
            pltpu.VMEM((CH,), jnp.int32),
            pltpu.VMEM((CH, D), jnp.float32),
            pltpu.VMEM((CH, D), jnp.float32),
            pltpu.VMEM((LANES,), jnp.float32),
            pltpu.VMEM((LANES,), jnp.float32),
            pltpu.SemaphoreType.DMA,
            pltpu.SemaphoreType.DMA,
        ],
    )
    def _gather_combine(ct_hbm, lt_hbm, idx_hbm, a_hbm, b_hbm, y_hbm,
                        idx_v, rc, rl, a_v, b_v, sem_c, sem_l):
        wid = lax.axis_index("s") * NC + lax.axis_index("c")
        base0 = wid * PER_W
        pltpu.sync_copy(a_hbm, a_v)
        pltpu.sync_copy(b_hbm, b_v)
        alpha = a_v[...]
        beta = b_v[...]

        def chunk(j, carry):
            base = base0 + j * CH
            pltpu.sync_copy(idx_hbm.at[pl.ds(base, CH)], idx_v)
            cp_c = pltpu.async_copy(ct_hbm.at[idx_v], rc, sem_c)
            cp_l = pltpu.async_copy(lt_hbm.at[idx_v], rl, sem_l)
            cp_c.wait()
            cp_l.wait()

            def row(r, c2):
                for q in range(D // LANES):
                    s = pl.ds(q * LANES, LANES)
                    rc[r, s] = rc[r, s] * alpha + rl[r, s] * beta
                return c2

            lax.fori_loop(0, CH, row, 0)
            pltpu.sync_copy(rc, y_hbm.at[pl.ds(base, CH)])
            return carry

        lax.fori_loop(0, NCH, chunk, 0)

    return _gather_combine


BLK = 1024
H = 4 * D


def _ffn_body(x_ref, y_ref, g_ref, bb_ref, w1_ref, b1_ref, w2_ref, b2_ref,
              o_ref):
    xb = x_ref[...] + y_ref[...]
    mu = jnp.mean(xb, axis=1, keepdims=True)
    xc = xb - mu
    var = jnp.mean(xc * xc, axis=1, keepdims=True)
    h = xc * lax.rsqrt(var + 1e-5) * g_ref[...] + bb_ref[...]
    a = jnp.dot(h, w1_ref[...], preferred_element_type=jnp.float32) + b1_ref[...]
    a = a * 0.5 * (1.0 + lax.erf(a * 0.7071067811865476))
    f = jnp.dot(a, w2_ref[...], preferred_element_type=jnp.float32) + b2_ref[...]
    o_ref[...] = xb + f


_ffn = pl.pallas_call(
    _ffn_body,
    grid=(N // BLK,),
    in_specs=[
        pl.BlockSpec((BLK, D), lambda i: (i, 0)),
        pl.BlockSpec((BLK, D), lambda i: (i, 0)),
        pl.BlockSpec((1, D), lambda i: (0, 0)),
        pl.BlockSpec((1, D), lambda i: (0, 0)),
        pl.BlockSpec((D, H), lambda i: (0, 0)),
        pl.BlockSpec((1, H), lambda i: (0, 0)),
        pl.BlockSpec((H, D), lambda i: (0, 0)),
        pl.BlockSpec((1, D), lambda i: (0, 0)),
    ],
    out_specs=pl.BlockSpec((BLK, D), lambda i: (i, 0)),
    out_shape=jax.ShapeDtypeStruct((N, D), jnp.float32),
)


@jax.jit
def kernel(x, idx, concept_table, law_table, alpha, beta, ln_gamma, ln_beta,
           W1, b1, W2, b2):
    xf = x.reshape(N, D)
    idxf = idx.reshape(N).astype(jnp.int32)
    a_vec = jnp.full((LANES,), alpha, jnp.float32)
    b_vec = jnp.full((LANES,), beta, jnp.float32)
    y = _make_gather_combine()(concept_table, law_table, idxf, a_vec, b_vec)
    out = _ffn(xf, y, ln_gamma.reshape(1, D), ln_beta.reshape(1, D),
               W1, b1.reshape(1, H), W2, b2.reshape(1, D))
    return out.reshape(B, L, D)


# speedup vs baseline: 1.0854x; 1.0854x over previous
import functools

import jax
import jax.numpy as jnp
from jax import lax
from jax.experimental import pallas as pl
from jax.experimental.pallas import tpu as pltpu
from jax.experimental.pallas import tpu_sc as plsc

B, L, D, V = 1024, 200, 64, 1000000
N = B * L
NC, NS = 2, 16
NW = NC * NS
PER_W = N // NW
CH = 128
NCH = PER_W // CH
DF = 2 * D


@functools.lru_cache(maxsize=1)
def _make_gather():
    mesh = plsc.VectorSubcoreMesh(
        core_axis_name="c", subcore_axis_name="s",
        num_cores=NC, num_subcores=NS)

    @functools.partial(
        pl.kernel,
        mesh=mesh,
        out_type=jax.ShapeDtypeStruct((N, DF), jnp.float32),
        scratch_types=[
            pltpu.VMEM((CH,), jnp.int32),
            pltpu.VMEM((CH,), jnp.int32),
            pltpu.VMEM((CH, DF), jnp.float32),
            pltpu.VMEM((CH, DF), jnp.float32),
            pltpu.SemaphoreType.DMA,
            pltpu.SemaphoreType.DMA,
        ],
    )
    def _gather(tbl_hbm, idx_hbm, g_hbm, idx0, idx1, buf0, buf1, sem0, sem1):
        wid = lax.axis_index("s") * NC + lax.axis_index("c")
        base0 = wid * PER_W

        idxs = (idx0, idx1)
        bufs = (buf0, buf1)
        sems = (sem0, sem1)

        def fire(j, b):
            pltpu.sync_copy(idx_hbm.at[pl.ds(base0 + j * CH, CH)], idxs[b])
            return pltpu.async_copy(tbl_hbm.at[idxs[b]], bufs[b], sems[b])

        fire(0, 0).wait()

        def pair(o, carry):
            for b in range(2):
                j = 2 * o + b

                @pl.when(j + 1 < NCH)
                def _():
                    fire(j + 1, 1 - b)

                @pl.when(j > 0)
                def _():
                    pltpu.make_async_copy(
                        tbl_hbm.at[idxs[b]], bufs[b], sems[b]).wait()

                pltpu.sync_copy(bufs[b],
                                g_hbm.at[pl.ds(base0 + j * CH, CH)])
            return carry

        lax.fori_loop(0, NCH // 2, pair, 0)

    return _gather


BLK = 1024
H = 4 * D


def _ffn_body(x_ref, g_ref, s_ref, gm_ref, bb_ref, w1_ref, b1_ref, w2_ref,
              b2_ref, o_ref):
    gs = g_ref[...] * s_ref[...]
    xb = x_ref[...] + gs[:, :D] + gs[:, D:]
    mu = jnp.mean(xb, axis=1, keepdims=True)
    xc = xb - mu
    var = jnp.mean(xc * xc, axis=1, keepdims=True)
    h = xc * lax.rsqrt(var + 1e-5) * gm_ref[...] + bb_ref[...]
    a = jnp.dot(h, w1_ref[...], preferred_element_type=jnp.float32) + b1_ref[...]
    a = a * 0.5 * (1.0 + lax.erf(a * 0.7071067811865476))
    f = jnp.dot(a, w2_ref[...], preferred_element_type=jnp.float32) + b2_ref[...]
    o_ref[...] = xb + f


_ffn = pl.pallas_call(
    _ffn_body,
    grid=(N // BLK,),
    in_specs=[
        pl.BlockSpec((BLK, D), lambda i: (i, 0)),
        pl.BlockSpec((BLK, DF), lambda i: (i, 0)),
        pl.BlockSpec((1, DF), lambda i: (0, 0)),
        pl.BlockSpec((1, D), lambda i: (0, 0)),
        pl.BlockSpec((1, D), lambda i: (0, 0)),
        pl.BlockSpec((D, H), lambda i: (0, 0)),
        pl.BlockSpec((1, H), lambda i: (0, 0)),
        pl.BlockSpec((H, D), lambda i: (0, 0)),
        pl.BlockSpec((1, D), lambda i: (0, 0)),
    ],
    out_specs=pl.BlockSpec((BLK, D), lambda i: (i, 0)),
    out_shape=jax.ShapeDtypeStruct((N, D), jnp.float32),
)


@jax.jit
def kernel(x, idx, concept_table, law_table, alpha, beta, ln_gamma, ln_beta,
           W1, b1, W2, b2):
    xf = x.reshape(N, D)
    idxf = idx.reshape(N).astype(jnp.int32)
    tbl = jnp.concatenate([concept_table, law_table], axis=1)
    g = _make_gather()(tbl, idxf)
    s = jnp.concatenate([jnp.full((D,), alpha, jnp.float32),
                         jnp.full((D,), beta, jnp.float32)]).reshape(1, DF)
    out = _ffn(xf, g, s, ln_gamma.reshape(1, D), ln_beta.reshape(1, D),
               W1, b1.reshape(1, H), W2, b2.reshape(1, D))
    return out.reshape(B, L, D)
